# baseline (speedup 1.0000x reference)
import jax
import jax.numpy as jnp
from jax import lax
from jax.experimental import pallas as pl
from jax.experimental.pallas import tpu as pltpu
from jax.experimental.pallas import tpu_sc as plsc

N_NODES = 10000
N_EDGES = 160000
D = 256

NC = 2
NS = 16
HALF = D // NC
EPT = N_EDGES // NS
CHUNK = 128
NCHUNK = EPT // CHUNK
TAIL = EPT - NCHUNK * CHUNK
ROWS_A = 624
BLK_A, NBLK_A = 24, 26
ROWS_LAST = N_NODES - ROWS_A * (NS - 1)


def _sc_scatter_body(ft_hbm, idx_hbm,
                     sums_hbm, counts_hbm,
                     idx1_v, vals_v, idx2_v, vals2_v,
                     idxt_v, ones_v, rowbuf_v, zvec_v,
                     ls0, ls1, ss0, ss1,
                     acc_sh, cnt_sh):
    c = lax.axis_index("c")
    s = lax.axis_index("s")
    r0 = s * ROWS_A
    col0 = c * HALF

    def load_copies(g, idx_b, vals_b, sem):
        base = s * EPT + g * CHUNK
        return (
            pltpu.make_async_copy(idx_hbm.at[pl.ds(base, CHUNK)], idx_b, sem),
            pltpu.make_async_copy(
                ft_hbm.at[pl.ds(base, CHUNK), pl.ds(col0, HALF)], vals_b, sem),
        )

    for cp in load_copies(0, idx1_v, vals_v, ls0):
        cp.start()

    for i in range(BLK_A):
        for t in range(HALF // 16):
            rowbuf_v[i, pl.ds(t * 16, 16)] = jnp.zeros((16,), jnp.float32)
    init_cps = [
        pltpu.make_async_copy(rowbuf_v.at[pl.ds(0, BLK_A), :],
                              acc_sh.at[pl.ds(r0 + k * BLK_A, BLK_A), :], ss0)
        for k in range(NBLK_A)
    ]
    for cp in init_cps:
        cp.start()
    for cp in init_cps:
        cp.wait()

    for i in range(ROWS_LAST // 16):
        zvec_v[pl.ds(i * 16, 16)] = jnp.zeros((16,), jnp.float32)
    for i in range(CHUNK // 16):
        ones_v[pl.ds(i * 16, 16)] = jnp.ones((16,), jnp.float32)

    @pl.when(s < NS - 1)
    def _():
        pltpu.sync_copy(zvec_v.at[pl.ds(0, ROWS_A)], cnt_sh.at[pl.ds(r0, ROWS_A)])

    @pl.when(s == NS - 1)
    def _():
        pltpu.sync_copy(rowbuf_v.at[pl.ds(0, ROWS_LAST - ROWS_A), :],
                        acc_sh.at[pl.ds(ROWS_A * NS, ROWS_LAST - ROWS_A), :])
        pltpu.sync_copy(zvec_v, cnt_sh.at[pl.ds(r0, ROWS_LAST)])

    plsc.subcore_barrier()

    def scatter_drain(idx_b, vals_b, ssem):
        pltpu.make_async_copy(vals_b, acc_sh.at[idx_b], ssem).wait()

        @pl.when(c == 0)
        def _():
            pltpu.make_async_copy(ones_v, cnt_sh.at[idx_b], ssem).wait()

    def step(g, idx_b, vals_b, sem_b, ss_b, idx_o, vals_o, sem_o, ss_o,
             first=False):
        for cp in load_copies(g, idx_b, vals_b, sem_b):
            cp.wait()

        if not first:
            scatter_drain(idx_o, vals_o, ss_o)

        @pl.when(g + 1 < NCHUNK)
        def _():
            for cp in load_copies(g + 1, idx_o, vals_o, sem_o):
                cp.start()

        pltpu.async_copy(vals_b, acc_sh.at[idx_b], ss_b, add=True)

        @pl.when(c == 0)
        def _():
            pltpu.async_copy(ones_v, cnt_sh.at[idx_b], ss_b, add=True)

    def pair_body(p, carry):
        g = p * 2

        @pl.when(p == 0)
        def _():
            step(g, idx1_v, vals_v, ls0, ss0, idx2_v, vals2_v, ls1, ss1,
                 first=True)

        @pl.when(p > 0)
        def _():
            step(g, idx1_v, vals_v, ls0, ss0, idx2_v, vals2_v, ls1, ss1)

        step(g + 1, idx2_v, vals2_v, ls1, ss1, idx1_v, vals_v, ls0, ss0)
        return carry

    lax.fori_loop(0, NCHUNK // 2, pair_body, 0)
    tbase = s * EPT + NCHUNK * CHUNK
    pltpu.sync_copy(idx_hbm.at[pl.ds(tbase, TAIL)], idxt_v)
    pltpu.sync_copy(ft_hbm.at[pl.ds(tbase, TAIL), pl.ds(col0, HALF)],
                    vals_v.at[pl.ds(0, TAIL), :])
    scatter_drain(idx2_v, vals2_v, ss1)
    pltpu.sync_copy(vals_v.at[pl.ds(0, TAIL), :], acc_sh.at[idxt_v], add=True)

    @pl.when(c == 0)
    def _():
        pltpu.sync_copy(ones_v.at[pl.ds(0, TAIL)], cnt_sh.at[idxt_v],
                        add=True)

    plsc.subcore_barrier()

    @pl.when(s < NS - 1)
    def _():
        pltpu.sync_copy(acc_sh.at[pl.ds(r0, ROWS_A), :],
                        sums_hbm.at[c, pl.ds(r0, ROWS_A), :])

        @pl.when(c == 0)
        def _():
            pltpu.sync_copy(cnt_sh.at[pl.ds(r0, ROWS_A)],
                            zvec_v.at[pl.ds(0, ROWS_A)])
            pltpu.sync_copy(zvec_v.at[pl.ds(0, ROWS_A)],
                            counts_hbm.at[pl.ds(r0, ROWS_A)])

    @pl.when(s == NS - 1)
    def _():
        pltpu.sync_copy(acc_sh.at[pl.ds(r0, ROWS_LAST), :],
                        sums_hbm.at[c, pl.ds(r0, ROWS_LAST), :])

        @pl.when(c == 0)
        def _():
            pltpu.sync_copy(cnt_sh.at[pl.ds(r0, ROWS_LAST)], zvec_v)
            pltpu.sync_copy(zvec_v, counts_hbm.at[pl.ds(r0, ROWS_LAST)])


def _sc_scatter(from_tensor, to_index):
    mesh = plsc.VectorSubcoreMesh(core_axis_name="c", subcore_axis_name="s")
    fn = pl.kernel(
        _sc_scatter_body,
        out_type=(jax.ShapeDtypeStruct((NC, N_NODES, HALF), jnp.float32),
                  jax.ShapeDtypeStruct((N_NODES,), jnp.float32)),
        mesh=mesh,
        scratch_types=[
            pltpu.VMEM((CHUNK,), jnp.int32),
            pltpu.VMEM((CHUNK, HALF), jnp.float32),
            pltpu.VMEM((CHUNK,), jnp.int32),
            pltpu.VMEM((CHUNK, HALF), jnp.float32),
            pltpu.VMEM((TAIL,), jnp.int32),
            pltpu.VMEM((CHUNK,), jnp.float32),
            pltpu.VMEM((BLK_A, HALF), jnp.float32),
            pltpu.VMEM((ROWS_LAST,), jnp.float32),
            pltpu.SemaphoreType.DMA,
            pltpu.SemaphoreType.DMA,
            pltpu.SemaphoreType.DMA,
            pltpu.SemaphoreType.DMA,
            pltpu.VMEM_SHARED((N_NODES, HALF), jnp.float32),
            pltpu.VMEM_SHARED((N_NODES,), jnp.float32),
        ],
    )
    return fn(from_tensor, to_index)


BLK = 10000


def _mlp_body(sums_ref, cnt_ref, w1_ref, b1_ref, w2_ref, b2_ref, out_ref):
    inv = 1.0 / jnp.maximum(cnt_ref[:, 0:1], 1.0)
    m0 = (sums_ref[0] * inv).astype(jnp.bfloat16)
    m1 = (sums_ref[1] * inv).astype(jnp.bfloat16)
    w1 = w1_ref[...].astype(jnp.bfloat16)
    h = (jnp.dot(m0, w1[0:HALF, :], preferred_element_type=jnp.float32)
         + jnp.dot(m1, w1[HALF:D, :], preferred_element_type=jnp.float32)
         + b1_ref[0:1, :])
    h = jnp.maximum(h, 0.0).astype(jnp.bfloat16)
    h = (jnp.dot(h, w2_ref[...].astype(jnp.bfloat16),
                 preferred_element_type=jnp.float32)
         + b2_ref[0:1, :])
    out_ref[...] = jnp.maximum(h, 0.0)


def _mlp(sums, counts, W1, b1, W2, b2):
    return pl.pallas_call(
        _mlp_body,
        grid=(N_NODES // BLK,),
        in_specs=[
            pl.BlockSpec((NC, BLK, HALF), lambda i: (0, i, 0)),
            pl.BlockSpec((BLK, 16), lambda i: (i, 0)),
            pl.BlockSpec((D, D), lambda i: (0, 0)),
            pl.BlockSpec((1, D), lambda i: (0, 0)),
            pl.BlockSpec((D, D), lambda i: (0, 0)),
            pl.BlockSpec((1, D), lambda i: (0, 0)),
        ],
        out_specs=pl.BlockSpec((BLK, D), lambda i: (i, 0)),
        out_shape=jax.ShapeDtypeStruct((N_NODES, D), jnp.float32),
    )(sums, counts, W1, b1, W2, b2)


def kernel(from_tensor, to_index, dim_size, W1, b1, W2, b2):
    sums, counts = _sc_scatter(from_tensor, to_index)
    counts2d = jnp.broadcast_to(counts[:, None], (N_NODES, 16))
    return _mlp(sums, counts2d, W1, b1.reshape(1, D), W2, b2.reshape(1, D))


# speedup vs baseline: 1.0194x; 1.0194x over previous
import jax
import jax.numpy as jnp
from jax import lax
from jax.experimental import pallas as pl
from jax.experimental.pallas import tpu as pltpu
from jax.experimental.pallas import tpu_sc as plsc

N_NODES = 10000
N_EDGES = 160000
D = 256

NC = 2
NS = 16
HALF = D // NC
EPT = N_EDGES // NS
CHUNK = 128
NCHUNK = EPT // CHUNK
TAIL = EPT - NCHUNK * CHUNK
ROWS_A = 624
BLK_A, NBLK_A = 24, 26
ROWS_LAST = N_NODES - ROWS_A * (NS - 1)


def _sc_scatter_body(ft_hbm, idx_hbm,
                     sums_hbm, counts_hbm,
                     idx1_v, vals_v, idx2_v, vals2_v,
                     idxt_v, ones_v, rowbuf_v, zvec_v,
                     ls0, ls1, ss0, ss1,
                     acc_sh, cnt_sh):
    c = lax.axis_index("c")
    s = lax.axis_index("s")
    r0 = s * ROWS_A
    col0 = c * HALF

    def load_copies(g, idx_b, vals_b, sem):
        base = s * EPT + g * CHUNK
        return (
            pltpu.make_async_copy(idx_hbm.at[pl.ds(base, CHUNK)], idx_b, sem),
            pltpu.make_async_copy(
                ft_hbm.at[pl.ds(base, CHUNK), pl.ds(col0, HALF)], vals_b, sem),
        )

    for cp in load_copies(0, idx1_v, vals_v, ls0):
        cp.start()

    for i in range(BLK_A):
        for t in range(HALF // 16):
            rowbuf_v[i, pl.ds(t * 16, 16)] = jnp.zeros((16,), jnp.float32)
    init_cps = [
        pltpu.make_async_copy(rowbuf_v.at[pl.ds(0, BLK_A), :],
                              acc_sh.at[pl.ds(r0 + k * BLK_A, BLK_A), :], ss0)
        for k in range(NBLK_A)
    ]
    for cp in init_cps:
        cp.start()
    for cp in init_cps:
        cp.wait()

    for i in range(ROWS_LAST // 16):
        zvec_v[pl.ds(i * 16, 16)] = jnp.zeros((16,), jnp.float32)
    for i in range(CHUNK // 16):
        ones_v[pl.ds(i * 16, 16)] = jnp.ones((16,), jnp.float32)

    @pl.when(s < NS - 1)
    def _():
        pltpu.sync_copy(zvec_v.at[pl.ds(0, ROWS_A)], cnt_sh.at[pl.ds(r0, ROWS_A)])

    @pl.when(s == NS - 1)
    def _():
        pltpu.sync_copy(rowbuf_v.at[pl.ds(0, ROWS_LAST - ROWS_A), :],
                        acc_sh.at[pl.ds(ROWS_A * NS, ROWS_LAST - ROWS_A), :])
        pltpu.sync_copy(zvec_v, cnt_sh.at[pl.ds(r0, ROWS_LAST)])

    plsc.subcore_barrier()

    def scatter_drain(idx_b, vals_b, ssem):
        pltpu.make_async_copy(vals_b, acc_sh.at[idx_b], ssem).wait()

        @pl.when(c == 0)
        def _():
            pltpu.make_async_copy(ones_v, cnt_sh.at[idx_b], ssem).wait()

    def step(g, idx_b, vals_b, sem_b, ss_b, idx_o, vals_o, sem_o, ss_o,
             first=False):
        for cp in load_copies(g, idx_b, vals_b, sem_b):
            cp.wait()

        if not first:
            scatter_drain(idx_o, vals_o, ss_o)

        @pl.when(g + 1 < NCHUNK)
        def _():
            for cp in load_copies(g + 1, idx_o, vals_o, sem_o):
                cp.start()

        pltpu.async_copy(vals_b, acc_sh.at[idx_b], ss_b, add=True)

        @pl.when(c == 0)
        def _():
            pltpu.async_copy(ones_v, cnt_sh.at[idx_b], ss_b, add=True)

    def pair_body(p, carry):
        g = p * 2

        @pl.when(p == 0)
        def _():
            step(g, idx1_v, vals_v, ls0, ss0, idx2_v, vals2_v, ls1, ss1,
                 first=True)

        @pl.when(p > 0)
        def _():
            step(g, idx1_v, vals_v, ls0, ss0, idx2_v, vals2_v, ls1, ss1)

        step(g + 1, idx2_v, vals2_v, ls1, ss1, idx1_v, vals_v, ls0, ss0)
        return carry

    lax.fori_loop(0, NCHUNK // 2, pair_body, 0)
    tbase = s * EPT + NCHUNK * CHUNK
    pltpu.sync_copy(idx_hbm.at[pl.ds(tbase, TAIL)], idxt_v)
    pltpu.sync_copy(ft_hbm.at[pl.ds(tbase, TAIL), pl.ds(col0, HALF)],
                    vals_v.at[pl.ds(0, TAIL), :])
    scatter_drain(idx2_v, vals2_v, ss1)
    pltpu.sync_copy(vals_v.at[pl.ds(0, TAIL), :], acc_sh.at[idxt_v], add=True)

    @pl.when(c == 0)
    def _():
        pltpu.sync_copy(ones_v.at[pl.ds(0, TAIL)], cnt_sh.at[idxt_v],
                        add=True)

    plsc.subcore_barrier()

    @pl.when(s < NS - 1)
    def _():
        pltpu.sync_copy(acc_sh.at[pl.ds(r0, ROWS_A), :],
                        sums_hbm.at[c, pl.ds(r0, ROWS_A), :])

        @pl.when(c == 0)
        def _():
            pltpu.sync_copy(cnt_sh.at[pl.ds(r0, ROWS_A)],
                            zvec_v.at[pl.ds(0, ROWS_A)])
            pltpu.sync_copy(zvec_v.at[pl.ds(0, ROWS_A)],
                            counts_hbm.at[pl.ds(r0, ROWS_A)])

    @pl.when(s == NS - 1)
    def _():
        pltpu.sync_copy(acc_sh.at[pl.ds(r0, ROWS_LAST), :],
                        sums_hbm.at[c, pl.ds(r0, ROWS_LAST), :])

        @pl.when(c == 0)
        def _():
            pltpu.sync_copy(cnt_sh.at[pl.ds(r0, ROWS_LAST)], zvec_v)
            pltpu.sync_copy(zvec_v, counts_hbm.at[pl.ds(r0, ROWS_LAST)])


def _sc_scatter(from_tensor, to_index):
    mesh = plsc.VectorSubcoreMesh(core_axis_name="c", subcore_axis_name="s")
    fn = pl.kernel(
        _sc_scatter_body,
        out_type=(jax.ShapeDtypeStruct((NC, N_NODES, HALF), jnp.float32),
                  jax.ShapeDtypeStruct((N_NODES,), jnp.float32)),
        mesh=mesh,
        scratch_types=[
            pltpu.VMEM((CHUNK,), jnp.int32),
            pltpu.VMEM((CHUNK, HALF), jnp.float32),
            pltpu.VMEM((CHUNK,), jnp.int32),
            pltpu.VMEM((CHUNK, HALF), jnp.float32),
            pltpu.VMEM((TAIL,), jnp.int32),
            pltpu.VMEM((CHUNK,), jnp.float32),
            pltpu.VMEM((BLK_A, HALF), jnp.float32),
            pltpu.VMEM((ROWS_LAST,), jnp.float32),
            pltpu.SemaphoreType.DMA,
            pltpu.SemaphoreType.DMA,
            pltpu.SemaphoreType.DMA,
            pltpu.SemaphoreType.DMA,
            pltpu.VMEM_SHARED((N_NODES, HALF), jnp.float32),
            pltpu.VMEM_SHARED((N_NODES,), jnp.float32),
        ],
    )
    return fn(from_tensor, to_index)


BLK = 5000


def _mlp_body(sums_ref, cnt_ref, w1_ref, b1_ref, w2_ref, b2_ref, out_ref):
    inv = 1.0 / jnp.maximum(cnt_ref[:, 0:1], 1.0)
    m0 = (sums_ref[0] * inv).astype(jnp.bfloat16)
    m1 = (sums_ref[1] * inv).astype(jnp.bfloat16)
    w1 = w1_ref[...].astype(jnp.bfloat16)
    h = (jnp.dot(m0, w1[0:HALF, :], preferred_element_type=jnp.float32)
         + jnp.dot(m1, w1[HALF:D, :], preferred_element_type=jnp.float32)
         + b1_ref[0:1, :])
    h = jnp.maximum(h, 0.0).astype(jnp.bfloat16)
    h = (jnp.dot(h, w2_ref[...].astype(jnp.bfloat16),
                 preferred_element_type=jnp.float32)
         + b2_ref[0:1, :])
    out_ref[...] = jnp.maximum(h, 0.0)


def _mlp(sums, counts, W1, b1, W2, b2):
    return pl.pallas_call(
        _mlp_body,
        grid=(N_NODES // BLK,),
        in_specs=[
            pl.BlockSpec((NC, BLK, HALF), lambda i: (0, i, 0)),
            pl.BlockSpec((BLK, 16), lambda i: (i, 0)),
            pl.BlockSpec((D, D), lambda i: (0, 0)),
            pl.BlockSpec((1, D), lambda i: (0, 0)),
            pl.BlockSpec((D, D), lambda i: (0, 0)),
            pl.BlockSpec((1, D), lambda i: (0, 0)),
        ],
        out_specs=pl.BlockSpec((BLK, D), lambda i: (i, 0)),
        out_shape=jax.ShapeDtypeStruct((N_NODES, D), jnp.float32),
    )(sums, counts, W1, b1, W2, b2)


def kernel(from_tensor, to_index, dim_size, W1, b1, W2, b2):
    sums, counts = _sc_scatter(from_tensor, to_index)
    counts2d = jnp.broadcast_to(counts[:, None], (N_NODES, 16))
    return _mlp(sums, counts2d, W1, b1.reshape(1, D), W2, b2.reshape(1, D))
